# Initial kernel scaffold; baseline (speedup 1.0000x reference)
import jax
import jax.numpy as jnp
from jax.experimental import pallas as pl


def kernel(hidden_states, W_router, W_gate, W_up, W_down, Ws_gate, Ws_up, Ws_down):
    raise NotImplementedError("write your pallas kernel here")


# speedup vs baseline: 2.3279x; 2.3279x over previous
import functools

import jax
import jax.numpy as jnp
from jax import lax
from jax.experimental import pallas as pl
from jax.experimental.pallas import tpu as pltpu
from jax.experimental.pallas import tpu_sc as plsc

T, D, F, E = 4096, 2048, 2048, 8
BT = 256
BF = 512
NF = F // BF
TP = T + E * BT
NT = TP // BT
NTS = T // BT


def _router_body(x_ref, wr_ref, eid_ref, xs_ref):
    x = x_ref[...]
    logits = jnp.dot(x, wr_ref[...], preferred_element_type=jnp.float32)
    mx = jnp.max(logits, axis=1, keepdims=True)
    cols = jax.lax.broadcasted_iota(jnp.int32, logits.shape, 1)
    eid = jnp.min(jnp.where(logits >= mx, cols, E), axis=1, keepdims=True)
    score = jax.nn.sigmoid(mx)
    eid_ref[...] = eid
    xs_ref[...] = x * score


def _router(x, w_router):
    return pl.pallas_call(
        _router_body,
        grid=(NTS,),
        in_specs=[
            pl.BlockSpec((BT, D), lambda t: (t, 0)),
            pl.BlockSpec((D, E), lambda t: (0, 0)),
        ],
        out_specs=[
            pl.BlockSpec((BT, 1), lambda t: (t, 0)),
            pl.BlockSpec((BT, D), lambda t: (t, 0)),
        ],
        out_shape=[
            jax.ShapeDtypeStruct((T, 1), jnp.int32),
            jax.ShapeDtypeStruct((T, D), jnp.float32),
        ],
    )(x, w_router)


def _meta_body(eid_ref, dst_ref, te_ref):
    e = eid_ref[...]
    lanes = jax.lax.broadcasted_iota(jnp.int32, (T, E), 1)
    onehot = (e == lanes).astype(jnp.float32)
    r = jax.lax.broadcasted_iota(jnp.int32, (BT, BT), 0)
    c = jax.lax.broadcasted_iota(jnp.int32, (BT, BT), 1)
    tri = (r >= c).astype(jnp.float32)
    carry = jnp.zeros((1, E), jnp.float32)
    incl = []
    for b in range(NTS):
        oh = onehot[b * BT:(b + 1) * BT, :]
        within = jnp.dot(tri, oh, preferred_element_type=jnp.float32)
        incl.append(within + carry)
        carry = carry + within[BT - 1:BT, :]
    counts = carry
    ntiles = jnp.ceil(counts / BT)
    r8 = jax.lax.broadcasted_iota(jnp.int32, (E, E), 0)
    c8 = jax.lax.broadcasted_iota(jnp.int32, (E, E), 1)
    triu8 = (r8 <= c8).astype(jnp.float32)
    cum_incl = jnp.dot(ntiles, triu8, preferred_element_type=jnp.float32)
    pad_off = (cum_incl - ntiles) * BT
    for b in range(NTS):
        vals = pad_off + incl[b] - 1.0
        oh = onehot[b * BT:(b + 1) * BT, :]
        dstb = jnp.sum(oh * vals, axis=1, keepdims=True)
        dst_ref[b * BT:(b + 1) * BT, :] = dstb.astype(jnp.int32)
    tl = jax.lax.broadcasted_iota(jnp.int32, (NT, E), 0)
    te = jnp.sum((tl >= cum_incl.astype(jnp.int32)).astype(jnp.int32),
                 axis=1, keepdims=True)
    te_ref[...] = jnp.minimum(te, E - 1)


def _routing_meta(eid):
    dst, te = pl.pallas_call(
        _meta_body,
        out_shape=[
            jax.ShapeDtypeStruct((T, 1), jnp.int32),
            jax.ShapeDtypeStruct((NT, 1), jnp.int32),
        ],
    )(eid)
    return dst, te[:, 0]


def _stage1_body(te_ref, xs_ref, wg_ref, wu_ref, h_ref):
    x = xs_ref[...]
    g = jnp.dot(x, wg_ref[0], preferred_element_type=jnp.float32)
    u = jnp.dot(x, wu_ref[0], preferred_element_type=jnp.float32)
    h_ref[...] = g * jax.nn.sigmoid(g) * u


def _stage1_routed(xs, w_gate, w_up, te):
    grid_spec = pltpu.PrefetchScalarGridSpec(
        num_scalar_prefetch=1,
        grid=(NF, NT),
        in_specs=[
            pl.BlockSpec((BT, D), lambda f, t, te: (t, 0)),
            pl.BlockSpec((1, D, BF), lambda f, t, te: (te[t], 0, f)),
            pl.BlockSpec((1, D, BF), lambda f, t, te: (te[t], 0, f)),
        ],
        out_specs=pl.BlockSpec((BT, BF), lambda f, t, te: (t, f)),
    )
    return pl.pallas_call(
        _stage1_body,
        grid_spec=grid_spec,
        out_shape=jax.ShapeDtypeStruct((TP, F), jnp.float32),
    )(te, xs, w_gate, w_up)


def _stage2_body(te_ref, h_ref, wd_ref, y_ref):
    y_ref[...] = jnp.dot(h_ref[...], wd_ref[0], preferred_element_type=jnp.float32)


def _stage2_routed(h, w_down, te):
    grid_spec = pltpu.PrefetchScalarGridSpec(
        num_scalar_prefetch=1,
        grid=(NT,),
        in_specs=[
            pl.BlockSpec((BT, F), lambda t, te: (t, 0)),
            pl.BlockSpec((1, F, D), lambda t, te: (te[t], 0, 0)),
        ],
        out_specs=pl.BlockSpec((BT, D), lambda t, te: (t, 0)),
    )
    return pl.pallas_call(
        _stage2_body,
        grid_spec=grid_spec,
        out_shape=jax.ShapeDtypeStruct((TP, D), jnp.float32),
    )(te, h, w_down)


def _shared1_body(x_ref, wg_ref, wu_ref, h_ref):
    x = x_ref[...]
    g = jnp.dot(x, wg_ref[...], preferred_element_type=jnp.float32)
    u = jnp.dot(x, wu_ref[...], preferred_element_type=jnp.float32)
    h_ref[...] = g * jax.nn.sigmoid(g) * u


def _shared1(x, ws_gate, ws_up):
    return pl.pallas_call(
        _shared1_body,
        grid=(NF, NTS),
        in_specs=[
            pl.BlockSpec((BT, D), lambda f, t: (t, 0)),
            pl.BlockSpec((D, BF), lambda f, t: (0, f)),
            pl.BlockSpec((D, BF), lambda f, t: (0, f)),
        ],
        out_specs=pl.BlockSpec((BT, BF), lambda f, t: (t, f)),
        out_shape=jax.ShapeDtypeStruct((T, F), jnp.float32),
    )(x, ws_gate, ws_up)


def _shared2_body(h_ref, wd_ref, r_ref, o_ref):
    o_ref[...] = (
        jnp.dot(h_ref[...], wd_ref[...], preferred_element_type=jnp.float32)
        + r_ref[...]
    )


def _shared2(h, ws_down, routed):
    return pl.pallas_call(
        _shared2_body,
        grid=(NTS,),
        in_specs=[
            pl.BlockSpec((BT, F), lambda t: (t, 0)),
            pl.BlockSpec((F, D), lambda t: (0, 0)),
            pl.BlockSpec((BT, D), lambda t: (t, 0)),
        ],
        out_specs=pl.BlockSpec((BT, D), lambda t: (t, 0)),
        out_shape=jax.ShapeDtypeStruct((T, D), jnp.float32),
    )(h, ws_down, routed)


NC, NS = 2, 16
NW = NC * NS
TOK_PER_W = T // NW
CH = 16
NCH = TOK_PER_W // CH


def _sc_mesh():
    return plsc.VectorSubcoreMesh(core_axis_name="c", subcore_axis_name="s")


def _dispatch_body(xsc_hbm, dst_hbm, xs_hbm, idx_v, buf_v, sem):
    wid = lax.axis_index("s") * NC + lax.axis_index("c")
    base = wid * TOK_PER_W
    pltpu.sync_copy(dst_hbm.at[wid], idx_v)

    def chunk(c, carry):
        pltpu.sync_copy(xsc_hbm.at[pl.ds(base + c * CH, CH)], buf_v)
        pltpu.async_copy(buf_v, xs_hbm.at[idx_v.at[c]], sem).wait()
        return carry

    lax.fori_loop(0, NCH, chunk, 0)


def _sc_dispatch(xscaled, dst3):
    fn = functools.partial(
        pl.kernel,
        mesh=_sc_mesh(),
        out_type=jax.ShapeDtypeStruct((TP, D), jnp.float32),
        scratch_types=[
            pltpu.VMEM((NCH, CH), jnp.int32),
            pltpu.VMEM((CH, D), jnp.float32),
            pltpu.SemaphoreType.DMA,
        ],
    )(_dispatch_body)
    return fn(xscaled, dst3)


def _combine_body(ys_hbm, dst_hbm, routed_hbm, idx_v, buf_v, sem):
    wid = lax.axis_index("s") * NC + lax.axis_index("c")
    base = wid * TOK_PER_W
    pltpu.sync_copy(dst_hbm.at[wid], idx_v)

    def chunk(c, carry):
        pltpu.async_copy(ys_hbm.at[idx_v.at[c]], buf_v, sem).wait()
        pltpu.sync_copy(buf_v, routed_hbm.at[pl.ds(base + c * CH, CH)])
        return carry

    lax.fori_loop(0, NCH, chunk, 0)


def _sc_combine(ys, dst3):
    fn = functools.partial(
        pl.kernel,
        mesh=_sc_mesh(),
        out_type=jax.ShapeDtypeStruct((T, D), jnp.float32),
        scratch_types=[
            pltpu.VMEM((NCH, CH), jnp.int32),
            pltpu.VMEM((CH, D), jnp.float32),
            pltpu.SemaphoreType.DMA,
        ],
    )(_combine_body)
    return fn(ys, dst3)


def kernel(hidden_states, W_router, W_gate, W_up, W_down, Ws_gate, Ws_up, Ws_down):
    eid, xscaled = _router(hidden_states, W_router)
    dst, te = _routing_meta(eid)
    dst3 = dst.reshape(NW, NCH, CH)
    xs = _sc_dispatch(xscaled, dst3)
    h = _stage1_routed(xs, W_gate, W_up, te)
    ys = _stage2_routed(h, W_down, te)
    routed = _sc_combine(ys, dst3)
    hs = _shared1(hidden_states, Ws_gate, Ws_up)
    return _shared2(hs, Ws_down, routed)
